# baseline (speedup 1.0000x reference)
import jax
import jax.numpy as jnp
import numpy as np
from jax import lax
from jax.experimental import pallas as pl
from jax.experimental.pallas import tpu as pltpu
from jax.experimental.pallas import tpu_sc as plsc

_N = 20000
_C = 76
_NW = 32
_RW = 640
_H1 = 256
_MAIN = 19968
_TAIL = _N - _MAIN
_GROUPS = _RW // 16

_F32 = jnp.float32
_I32 = jnp.int32


def _trunc(x):
    return x.astype(_I32).astype(_F32)


def _floor(x):
    f = _trunc(x)
    return jnp.where(f > x, f - np.float32(1.0), f)


def _sincos(r):
    half = jnp.where(r >= 0, np.float32(0.5), np.float32(-0.5))
    jf = _trunc(r * np.float32(2.0 / np.pi) + half)
    j = jf.astype(_I32)
    t = r - jf * np.float32(1.5703125)
    t = t - jf * np.float32(4.837512969970703e-4)
    t = t - jf * np.float32(7.54978995489188e-8)
    z = t * t
    sin_t = t + t * z * (np.float32(-1.6666654611e-1)
                         + z * (np.float32(8.3321608736e-3)
                                + z * np.float32(-1.9515295891e-4)))
    cos_t = np.float32(1.0) + z * (np.float32(-0.5)
                                   + z * (np.float32(4.166664568298827e-2)
                                          + z * (np.float32(-1.388731625493765e-3)
                                                 + z * np.float32(2.443315711809948e-5))))
    q = j & 3
    swap = (q & 1) == 1
    sbase = jnp.where(swap, cos_t, sin_t)
    cbase = jnp.where(swap, sin_t, cos_t)
    sin_r = jnp.where(q >= 2, -sbase, sbase)
    cos_r = jnp.where((q == 1) | (q == 2), -cbase, cbase)
    return sin_r, cos_r


def _decode_rows(pred_v, roi_v, out_v, r_in, r_out, a0, a1, a2):
    def argmax_with_res(ch0, res0):
        bv = pred_v[ch0, pl.ds(r_in, 16)]
        br = pred_v[res0, pl.ds(r_in, 16)]
        bi = jnp.zeros((16,), _I32)
        for jj in range(1, 12):
            v = pred_v[ch0 + jj, pl.ds(r_in, 16)]
            rr = pred_v[res0 + jj, pl.ds(r_in, 16)]
            take = v > bv
            bv = jnp.where(take, v, bv)
            br = jnp.where(take, rr, br)
            bi = jnp.where(take, jj, bi)
        return bi, br

    x_bin, x_res = argmax_with_res(0, 24)
    z_bin, z_res = argmax_with_res(12, 36)
    ry_bin, ry_resn = argmax_with_res(49, 61)

    y_res = pred_v[48, pl.ds(r_in, 16)]
    s0 = pred_v[73, pl.ds(r_in, 16)]
    s1 = pred_v[74, pl.ds(r_in, 16)]
    s2 = pred_v[75, pl.ds(r_in, 16)]
    roi_x = roi_v[0, pl.ds(r_in, 16)]
    roi_y = roi_v[1, pl.ds(r_in, 16)]
    roi_z = roi_v[2, pl.ds(r_in, 16)]
    roi_ry = roi_v[6, pl.ds(r_in, 16)]

    pos_x = x_bin.astype(_F32) * np.float32(0.5) + np.float32(0.25 - 3.0) \
        + x_res * np.float32(0.5)
    pos_z = z_bin.astype(_F32) * np.float32(0.5) + np.float32(0.25 - 3.0) \
        + z_res * np.float32(0.5)
    pos_y = roi_y + y_res

    aps = np.float32(2.0 * np.pi / 12.0)
    v = ry_bin.astype(_F32) * aps + ry_resn * (aps * np.float32(0.5))
    twopi = np.float32(2.0 * np.pi)
    m = v - _floor(v * (np.float32(1.0) / twopi)) * twopi
    ry = jnp.where(m > np.float32(np.pi), m - twopi, m)

    h = s0 * a0 + a0
    w = s1 * a1 + a1
    l = s2 * a2 + a2

    sin_r, cos_r = _sincos(roi_ry)
    out_x = pos_x * cos_r + pos_z * sin_r + roi_x
    out_z = -pos_x * sin_r + pos_z * cos_r + roi_z
    out_ry = ry + roi_ry

    for ci, val in enumerate((out_x, pos_y, out_z, h, w, l, out_ry)):
        out_v[ci, pl.ds(r_out, 16)] = val


def _body(roi_hbm, pred_hbm, anchor_hbm, ptail_hbm, rtail_hbm,
          out_hbm, out_tail_hbm, pred_v, roi_v, out_v, anchor_v, sem1, sem2):
    c = lax.axis_index("c")
    s = lax.axis_index("s")
    wid = s * 2 + c
    base = wid * _RW

    pltpu.sync_copy(anchor_hbm, anchor_v)
    a0 = anchor_v[0, pl.ds(0, 16)]
    a1 = anchor_v[1, pl.ds(0, 16)]
    a2 = anchor_v[2, pl.ds(0, 16)]

    def group(g, carry):
        _decode_rows(pred_v, roi_v, out_v, g * 16, g * 16, a0, a1, a2)
        return carry

    @pl.when(wid < _NW - 1)
    def _run_main():
        cp1 = pltpu.async_copy(pred_hbm.at[:, pl.ds(base, _H1)],
                               pred_v.at[:, pl.ds(0, _H1)], sem1)
        cp2 = pltpu.async_copy(pred_hbm.at[:, pl.ds(base + _H1, _RW - _H1)],
                               pred_v.at[:, pl.ds(_H1, _RW - _H1)], sem2)
        pltpu.sync_copy(roi_hbm.at[:, pl.ds(base, _RW)], roi_v)
        cp1.wait()
        lax.fori_loop(0, _H1 // 16, group, 0, unroll=2)
        cp2.wait()
        lax.fori_loop(_H1 // 16, _GROUPS, group, 0, unroll=2)
        pltpu.sync_copy(out_v, out_hbm.at[:, pl.ds(base, _RW)])

    @pl.when(wid == _NW - 1)
    def _run_last():
        cp1 = pltpu.async_copy(pred_hbm.at[:, pl.ds(_MAIN - 128, 128)],
                               pred_v.at[:, pl.ds(0, 128)], sem1)
        pltpu.sync_copy(roi_hbm.at[:, pl.ds(_MAIN - 128, 128)],
                        roi_v.at[:, pl.ds(0, 128)])
        pltpu.sync_copy(ptail_hbm, pred_v.at[:, pl.ds(128, 128)])
        pltpu.sync_copy(rtail_hbm, roi_v.at[:, pl.ds(128, 128)])
        cp1.wait()
        lax.fori_loop(0, 10, group, 0, unroll=False)
        pltpu.sync_copy(out_v.at[:, pl.ds(0, 128)],
                        out_hbm.at[:, pl.ds(_MAIN - 128, 128)])
        pltpu.sync_copy(out_v.at[:, pl.ds(128, 128)], out_tail_hbm)


@jax.jit
def _decode(roi_t, pred_t, anchor_blk, pred_blk, roi_blk):
    mesh = plsc.VectorSubcoreMesh(core_axis_name="c", subcore_axis_name="s")
    run = pl.kernel(
        _body,
        mesh=mesh,
        compiler_params=pltpu.CompilerParams(
            needs_layout_passes=False, use_tc_tiling_on_sc=True,
            skip_device_barrier=True, disable_bounds_checks=True),
        out_type=(jax.ShapeDtypeStruct((7, _N), _F32),
                  jax.ShapeDtypeStruct((7, 128), _F32)),
        scratch_types=[
            pltpu.VMEM((_C, _RW), _F32),
            pltpu.VMEM((7, _RW), _F32),
            pltpu.VMEM((7, _RW), _F32),
            pltpu.VMEM((3, 16), _F32),
            pltpu.SemaphoreType.DMA,
            pltpu.SemaphoreType.DMA,
        ],
    )
    return run(roi_t, pred_t, anchor_blk, pred_blk, roi_blk)


def kernel(roi_box3d, pred_reg, anchor_size):
    roi_t = roi_box3d.T
    pred_t = pred_reg.T
    anchor_blk = jnp.broadcast_to(anchor_size[:, None].astype(_F32), (3, 16))
    pred_blk = jnp.pad(lax.slice(pred_t, (0, _MAIN), (_C, _N)),
                       ((0, 0), (0, 128 - _TAIL)))
    roi_blk = jnp.pad(lax.slice(roi_t, (0, _MAIN), (7, _N)),
                      ((0, 0), (0, 128 - _TAIL)))
    out_t, out_tail = _decode(roi_t, pred_t, anchor_blk, pred_blk, roi_blk)
    out_t = lax.dynamic_update_slice(
        out_t, lax.slice(out_tail, (0, 0), (7, _TAIL)), (0, _MAIN))
    return out_t.T


# speedup vs baseline: 1.0309x; 1.0309x over previous
import jax
import jax.numpy as jnp
import numpy as np
from jax import lax
from jax.experimental import pallas as pl
from jax.experimental.pallas import tpu as pltpu
from jax.experimental.pallas import tpu_sc as plsc

_N = 20000
_C = 76
_NW = 32
_RW = 640
_H1 = 256
_MAIN = 19968
_TAIL = _N - _MAIN
_GROUPS = _RW // 16

_F32 = jnp.float32
_I32 = jnp.int32


def _trunc(x):
    return x.astype(_I32).astype(_F32)


def _floor(x):
    f = _trunc(x)
    return jnp.where(f > x, f - np.float32(1.0), f)


def _sincos(r):
    half = jnp.where(r >= 0, np.float32(0.5), np.float32(-0.5))
    jf = _trunc(r * np.float32(2.0 / np.pi) + half)
    j = jf.astype(_I32)
    t = r - jf * np.float32(1.5703125)
    t = t - jf * np.float32(4.837512969970703e-4)
    t = t - jf * np.float32(7.54978995489188e-8)
    z = t * t
    sin_t = t + t * z * (np.float32(-1.6666654611e-1)
                         + z * (np.float32(8.3321608736e-3)
                                + z * np.float32(-1.9515295891e-4)))
    cos_t = np.float32(1.0) + z * (np.float32(-0.5)
                                   + z * (np.float32(4.166664568298827e-2)
                                          + z * (np.float32(-1.388731625493765e-3)
                                                 + z * np.float32(2.443315711809948e-5))))
    q = j & 3
    swap = (q & 1) == 1
    sbase = jnp.where(swap, cos_t, sin_t)
    cbase = jnp.where(swap, sin_t, cos_t)
    sin_r = jnp.where(q >= 2, -sbase, sbase)
    cos_r = jnp.where((q == 1) | (q == 2), -cbase, cbase)
    return sin_r, cos_r


def _decode_rows(pred_v, roi_v, out_v, r_in, r_out, a0, a1, a2):
    def argmax_with_res(ch0, res0):
        bv = pred_v[ch0, pl.ds(r_in, 16)]
        br = pred_v[res0, pl.ds(r_in, 16)]
        bi = jnp.zeros((16,), _I32)
        for jj in range(1, 12):
            v = pred_v[ch0 + jj, pl.ds(r_in, 16)]
            rr = pred_v[res0 + jj, pl.ds(r_in, 16)]
            take = v > bv
            bv = jnp.where(take, v, bv)
            br = jnp.where(take, rr, br)
            bi = jnp.where(take, jj, bi)
        return bi, br

    x_bin, x_res = argmax_with_res(0, 24)
    z_bin, z_res = argmax_with_res(12, 36)
    ry_bin, ry_resn = argmax_with_res(49, 61)

    y_res = pred_v[48, pl.ds(r_in, 16)]
    s0 = pred_v[73, pl.ds(r_in, 16)]
    s1 = pred_v[74, pl.ds(r_in, 16)]
    s2 = pred_v[75, pl.ds(r_in, 16)]
    roi_x = roi_v[0, pl.ds(r_in, 16)]
    roi_y = roi_v[1, pl.ds(r_in, 16)]
    roi_z = roi_v[2, pl.ds(r_in, 16)]
    roi_ry = roi_v[6, pl.ds(r_in, 16)]

    pos_x = x_bin.astype(_F32) * np.float32(0.5) + np.float32(0.25 - 3.0) \
        + x_res * np.float32(0.5)
    pos_z = z_bin.astype(_F32) * np.float32(0.5) + np.float32(0.25 - 3.0) \
        + z_res * np.float32(0.5)
    pos_y = roi_y + y_res

    aps = np.float32(2.0 * np.pi / 12.0)
    v = ry_bin.astype(_F32) * aps + ry_resn * (aps * np.float32(0.5))
    twopi = np.float32(2.0 * np.pi)
    m = v - _floor(v * (np.float32(1.0) / twopi)) * twopi
    ry = jnp.where(m > np.float32(np.pi), m - twopi, m)

    h = s0 * a0 + a0
    w = s1 * a1 + a1
    l = s2 * a2 + a2

    sin_r, cos_r = _sincos(roi_ry)
    out_x = pos_x * cos_r + pos_z * sin_r + roi_x
    out_z = -pos_x * sin_r + pos_z * cos_r + roi_z
    out_ry = ry + roi_ry

    for ci, val in enumerate((out_x, pos_y, out_z, h, w, l, out_ry)):
        out_v[ci, pl.ds(r_out, 16)] = val


def _body(roi_hbm, pred_hbm, side_hbm, out_hbm, out_tail_hbm,
          pred_v, roi_v, out_v, anchor_v, ptail_v, rtail_v, sem1, sem2):
    c = lax.axis_index("c")
    s = lax.axis_index("s")
    wid = s * 2 + c
    base = wid * _RW

    pltpu.sync_copy(side_hbm.at[pl.ds(0, 8)], anchor_v)
    a0 = anchor_v[0, pl.ds(0, 16)]
    a1 = anchor_v[1, pl.ds(0, 16)]
    a2 = anchor_v[2, pl.ds(0, 16)]

    def group(g, carry):
        _decode_rows(pred_v, roi_v, out_v, g * 16, g * 16, a0, a1, a2)
        return carry

    @pl.when(wid < _NW - 1)
    def _run_main():
        cp1 = pltpu.async_copy(pred_hbm.at[:, pl.ds(base, _H1)],
                               pred_v.at[:, pl.ds(0, _H1)], sem1)
        cp2 = pltpu.async_copy(pred_hbm.at[:, pl.ds(base + _H1, _RW - _H1)],
                               pred_v.at[:, pl.ds(_H1, _RW - _H1)], sem2)
        pltpu.sync_copy(roi_hbm.at[:, pl.ds(base, _RW)], roi_v)
        cp1.wait()
        lax.fori_loop(0, _H1 // 16, group, 0, unroll=False)
        cp2.wait()
        lax.fori_loop(_H1 // 16, _GROUPS, group, 0, unroll=False)
        pltpu.sync_copy(out_v, out_hbm.at[:, pl.ds(base, _RW)])

    @pl.when(wid == _NW - 1)
    def _run_last():
        cp1 = pltpu.async_copy(pred_hbm.at[:, pl.ds(_MAIN - 128, 128)],
                               pred_v.at[:, pl.ds(0, 128)], sem1)
        pltpu.sync_copy(roi_hbm.at[:, pl.ds(_MAIN - 128, 128)],
                        roi_v.at[:, pl.ds(0, 128)])
        pltpu.sync_copy(side_hbm.at[pl.ds(8, 80)], ptail_v)
        pltpu.sync_copy(side_hbm.at[pl.ds(88, 8)], rtail_v)
        cp1.wait()
        lax.fori_loop(0, 8, group, 0, unroll=False)
        for gg in range(2):
            _decode_rows(ptail_v, rtail_v, out_v, gg * 16, 128 + gg * 16,
                         a0, a1, a2)
        pltpu.sync_copy(out_v.at[:, pl.ds(0, 128)],
                        out_hbm.at[:, pl.ds(_MAIN - 128, 128)])
        pltpu.sync_copy(out_v.at[:, pl.ds(128, 128)], out_tail_hbm)


@jax.jit
def _decode(roi_t, pred_t, side):
    mesh = plsc.VectorSubcoreMesh(core_axis_name="c", subcore_axis_name="s")
    run = pl.kernel(
        _body,
        mesh=mesh,
        compiler_params=pltpu.CompilerParams(
            needs_layout_passes=False, use_tc_tiling_on_sc=True,
            skip_device_barrier=True, disable_bounds_checks=True),
        out_type=(jax.ShapeDtypeStruct((7, _N), _F32),
                  jax.ShapeDtypeStruct((7, 128), _F32)),
        scratch_types=[
            pltpu.VMEM((_C, _RW), _F32),
            pltpu.VMEM((7, _RW), _F32),
            pltpu.VMEM((7, _RW), _F32),
            pltpu.VMEM((8, 128), _F32),
            pltpu.VMEM((80, 128), _F32),
            pltpu.VMEM((8, 128), _F32),
            pltpu.SemaphoreType.DMA,
            pltpu.SemaphoreType.DMA,
        ],
    )
    return run(roi_t, pred_t, side)


def kernel(roi_box3d, pred_reg, anchor_size):
    roi_t = roi_box3d.T
    pred_t = pred_reg.T
    anchor_blk = jnp.pad(jnp.broadcast_to(anchor_size[:, None].astype(_F32),
                                          (3, 128)), ((0, 5), (0, 0)))
    pred_blk = jnp.pad(lax.slice(pred_t, (0, _MAIN), (_C, _N)),
                       ((0, 4), (0, 128 - _TAIL)))
    roi_blk = jnp.pad(lax.slice(roi_t, (0, _MAIN), (7, _N)),
                      ((0, 1), (0, 128 - _TAIL)))
    side = jnp.concatenate([anchor_blk, pred_blk, roi_blk], axis=0)
    out_t, out_tail = _decode(roi_t, pred_t, side)
    out_t = lax.dynamic_update_slice(
        out_t, lax.slice(out_tail, (0, 0), (7, _TAIL)), (0, _MAIN))
    return out_t.T
